# Initial kernel scaffold; baseline (speedup 1.0000x reference)
import jax
import jax.numpy as jnp
from jax.experimental import pallas as pl


def kernel(inputs):
    raise NotImplementedError("write your pallas kernel here")


# speedup vs baseline: 6218.2264x; 6218.2264x over previous
import functools

import jax
import jax.numpy as jnp
from jax import lax
from jax.experimental import pallas as pl

EPS = 1e-7


def _ctc_kernel(x_ref, dec_ref, score_ref, *, T, V):
    x = x_ref[0]

    maxv = jnp.max(x, axis=1, keepdims=True)
    idx = lax.broadcasted_iota(jnp.int32, (T, V), 1)
    cand = jnp.where(x == maxv, idx, V)
    best = jnp.min(cand, axis=1, keepdims=True)

    score_ref[0] = jnp.sum(jnp.log(maxv + EPS)).reshape(1, 1)

    blank = V - 1
    prev = jnp.concatenate(
        [jnp.full((1, 1), -1, jnp.int32), best[:-1]], axis=0)
    keep = (best != prev) & (best != blank)

    c = keep.astype(jnp.float32)
    sh = 1
    while sh < T:
        c = c + jnp.concatenate(
            [jnp.zeros((sh, 1), jnp.float32), c[:-sh]], axis=0)
        sh *= 2
    pos = (c - 1.0).astype(jnp.int32)

    jidx = lax.broadcasted_iota(jnp.int32, (T, T), 1)
    onehot = ((pos == jidx) & keep).astype(jnp.float32)
    vals = (best + 1).astype(jnp.float32)
    dec_row = jnp.sum(onehot * vals, axis=0, keepdims=True) - 1.0
    dec_ref[0] = dec_row.astype(jnp.int32)


def kernel(inputs):
    B, T, V = inputs.shape
    grid = (B,)
    dec, score = pl.pallas_call(
        functools.partial(_ctc_kernel, T=T, V=V),
        grid=grid,
        in_specs=[pl.BlockSpec((1, T, V), lambda b: (b, 0, 0))],
        out_specs=[
            pl.BlockSpec((1, 1, T), lambda b: (b, 0, 0)),
            pl.BlockSpec((1, 1, 1), lambda b: (b, 0, 0)),
        ],
        out_shape=[
            jax.ShapeDtypeStruct((B, 1, T), jnp.int32),
            jax.ShapeDtypeStruct((B, 1, 1), jnp.float32),
        ],
    )(inputs)
    return dec.reshape(B, T), score.reshape(B, 1)
